# initial kernel scaffold (unmeasured)
import jax
import jax.numpy as jnp
from jax import lax
from jax.experimental import pallas as pl
from jax.experimental.pallas import tpu as pltpu

N_DEV = 4
N_LAYERS = 3
N_HOPS = N_DEV - 1


def kernel(x, Win0, Wout0, Win1, Wout1, Win2, Wout2):
    b, d_shard = x.shape
    _, hdim = Win0.shape

    def body(x_ref, win0_ref, wout0_ref, win1_ref, wout1_ref, win2_ref,
             wout2_ref, out_ref, comm_ref, send_sems, recv_sems):
        my_pos = lax.axis_index("i")
        left = (my_pos - 1) % N_DEV
        right = (my_pos + 1) % N_DEV

        barrier_sem = pltpu.get_barrier_semaphore()
        for nbr in (left, right):
            pl.semaphore_signal(
                barrier_sem, inc=1,
                device_id=(nbr,), device_id_type=pl.DeviceIdType.MESH,
            )
        pl.semaphore_wait(barrier_sem, 2)

        wins = [win0_ref, win1_ref, win2_ref]
        wouts = [wout0_ref, wout1_ref, wout2_ref]

        xv = x_ref[...]
        for l in range(N_LAYERS):
            partial = jnp.dot(xv, wins[l][...],
                              preferred_element_type=jnp.float32)
            own = 4 * l
            comm_ref[own] = partial
            acc = partial
            for h in range(N_HOPS):
                slot = 4 * l + 1 + h
                sem = N_HOPS * l + h
                src = comm_ref.at[own] if h == 0 else comm_ref.at[slot - 1]
                rdma = pltpu.make_async_remote_copy(
                    src_ref=src,
                    dst_ref=comm_ref.at[slot],
                    send_sem=send_sems.at[sem],
                    recv_sem=recv_sems.at[sem],
                    device_id=(right,),
                    device_id_type=pl.DeviceIdType.MESH,
                )
                rdma.start()
                rdma.wait()
                acc = acc + comm_ref[slot]
            hfull = jnp.maximum(acc, 0.0)
            xv = jnp.dot(hfull, wouts[l][...],
                         preferred_element_type=jnp.float32)
        out_ref[...] = xv

    return pl.pallas_call(
        body,
        out_shape=jax.ShapeDtypeStruct((b, d_shard), jnp.float32),
        in_specs=[pl.BlockSpec(memory_space=pltpu.VMEM)] * 7,
        out_specs=pl.BlockSpec(memory_space=pltpu.VMEM),
        scratch_shapes=[
            pltpu.VMEM((4 * N_LAYERS, b, hdim), jnp.float32),
            pltpu.SemaphoreType.DMA((N_LAYERS * N_HOPS,)),
            pltpu.SemaphoreType.DMA((N_LAYERS * N_HOPS,)),
        ],
        compiler_params=pltpu.CompilerParams(collective_id=0),
    )(x, Win0, Wout0, Win1, Wout1, Win2, Wout2)


# baseline (device time: 91770 ns/iter reference)
import functools

import jax
import jax.numpy as jnp
from jax import lax
from jax.experimental import pallas as pl
from jax.experimental.pallas import tpu as pltpu

N_DEV = 4
N_HOPS = N_DEV - 1


def _layer(x, win, wout, *, collective_id):
    b, d_shard = x.shape
    _, hdim = win.shape

    def body(x_ref, win_ref, wout_ref, out_ref, comm_ref, send_sems,
             recv_sems):
        my_pos = lax.axis_index("i")
        left = (my_pos - 1) % N_DEV
        right = (my_pos + 1) % N_DEV

        barrier_sem = pltpu.get_barrier_semaphore()
        for nbr in (left, right):
            pl.semaphore_signal(
                barrier_sem, inc=1,
                device_id=(nbr,), device_id_type=pl.DeviceIdType.MESH,
            )
        pl.semaphore_wait(barrier_sem, 2)

        partial = jnp.dot(x_ref[...], win_ref[...],
                          preferred_element_type=jnp.float32)
        comm_ref[0] = partial
        acc = partial
        for h in range(N_HOPS):
            slot = 1 + h
            rdma = pltpu.make_async_remote_copy(
                src_ref=comm_ref.at[slot - 1],
                dst_ref=comm_ref.at[slot],
                send_sem=send_sems.at[h],
                recv_sem=recv_sems.at[h],
                device_id=(right,),
                device_id_type=pl.DeviceIdType.MESH,
            )
            rdma.start()
            rdma.wait()
            acc = acc + comm_ref[slot]
        hfull = jnp.maximum(acc, 0.0)
        out_ref[...] = jnp.dot(hfull, wout_ref[...],
                               preferred_element_type=jnp.float32)

    return pl.pallas_call(
        body,
        out_shape=jax.ShapeDtypeStruct((b, d_shard), jnp.float32),
        in_specs=[pl.BlockSpec(memory_space=pltpu.VMEM)] * 3,
        out_specs=pl.BlockSpec(memory_space=pltpu.VMEM),
        scratch_shapes=[
            pltpu.VMEM((1 + N_HOPS, b, hdim), jnp.float32),
            pltpu.SemaphoreType.DMA((N_HOPS,)),
            pltpu.SemaphoreType.DMA((N_HOPS,)),
        ],
        compiler_params=pltpu.CompilerParams(collective_id=collective_id),
    )(x, win, wout)


def kernel(x, Win0, Wout0, Win1, Wout1, Win2, Wout2):
    x = _layer(x, Win0, Wout0, collective_id=0)
    x = _layer(x, Win1, Wout1, collective_id=1)
    x = _layer(x, Win2, Wout2, collective_id=2)
    return x


# device time: 56240 ns/iter; 1.6318x vs baseline; 1.6318x over previous
import jax
import jax.numpy as jnp
from jax import lax
from jax.experimental import pallas as pl
from jax.experimental.pallas import tpu as pltpu

N_DEV = 4


def _layer(x, win, wout, *, collective_id):
    b, d_shard = x.shape
    _, hdim = win.shape
    q = hdim // N_DEV

    def body(x_ref, win_ref, wout_ref, out_ref, pbuf, rs_buf, h_buf,
             rs_send, rs_recv, ag_send, ag_recv):
        my_pos = lax.axis_index("i")

        barrier_sem = pltpu.get_barrier_semaphore()
        for k in range(1, N_DEV):
            pl.semaphore_signal(
                barrier_sem, inc=1,
                device_id=((my_pos + k) % N_DEV,),
                device_id_type=pl.DeviceIdType.MESH,
            )
        pl.semaphore_wait(barrier_sem, N_DEV - 1)

        pbuf[...] = jnp.dot(x_ref[...], win_ref[...],
                            preferred_element_type=jnp.float32)

        rs = []
        for k in range(1, N_DEV):
            peer = (my_pos + k) % N_DEV
            rdma = pltpu.make_async_remote_copy(
                src_ref=pbuf.at[:, pl.ds(peer * q, q)],
                dst_ref=rs_buf.at[k - 1],
                send_sem=rs_send.at[k - 1],
                recv_sem=rs_recv.at[k - 1],
                device_id=(peer,),
                device_id_type=pl.DeviceIdType.MESH,
            )
            rdma.start()
            rs.append(rdma)
        for rdma in rs:
            rdma.wait()

        hq = pbuf[:, pl.ds(my_pos * q, q)]
        hq = hq + rs_buf[0] + rs_buf[1] + rs_buf[2]
        hq = jnp.maximum(hq, 0.0)
        h_buf[:, pl.ds(my_pos * q, q)] = hq

        ag = []
        for k in range(1, N_DEV):
            peer = (my_pos + k) % N_DEV
            rdma = pltpu.make_async_remote_copy(
                src_ref=h_buf.at[:, pl.ds(my_pos * q, q)],
                dst_ref=h_buf.at[:, pl.ds(my_pos * q, q)],
                send_sem=ag_send.at[k - 1],
                recv_sem=ag_recv.at[k - 1],
                device_id=(peer,),
                device_id_type=pl.DeviceIdType.MESH,
            )
            rdma.start()
            ag.append(rdma)
        for rdma in ag:
            rdma.wait()

        out_ref[...] = jnp.dot(h_buf[...], wout_ref[...],
                               preferred_element_type=jnp.float32)

    return pl.pallas_call(
        body,
        out_shape=jax.ShapeDtypeStruct((b, d_shard), jnp.float32),
        in_specs=[pl.BlockSpec(memory_space=pltpu.VMEM)] * 3,
        out_specs=pl.BlockSpec(memory_space=pltpu.VMEM),
        scratch_shapes=[
            pltpu.VMEM((b, hdim), jnp.float32),
            pltpu.VMEM((N_DEV - 1, b, q), jnp.float32),
            pltpu.VMEM((b, hdim), jnp.float32),
            pltpu.SemaphoreType.DMA((N_DEV - 1,)),
            pltpu.SemaphoreType.DMA((N_DEV - 1,)),
            pltpu.SemaphoreType.DMA((N_DEV - 1,)),
            pltpu.SemaphoreType.DMA((N_DEV - 1,)),
        ],
        compiler_params=pltpu.CompilerParams(collective_id=collective_id),
    )(x, win, wout)


def kernel(x, Win0, Wout0, Win1, Wout1, Win2, Wout2):
    x = _layer(x, Win0, Wout0, collective_id=0)
    x = _layer(x, Win1, Wout1, collective_id=1)
    x = _layer(x, Win2, Wout2, collective_id=2)
    return x


# device time: 55411 ns/iter; 1.6562x vs baseline; 1.0150x over previous
import jax
import jax.numpy as jnp
from jax import lax
from jax.experimental import pallas as pl
from jax.experimental.pallas import tpu as pltpu

N_DEV = 4


def _layer(x, win, wout, *, collective_id):
    b, d_shard = x.shape
    _, hdim = win.shape
    q = hdim // N_DEV

    def body(x_ref, win_ref, wout_ref, out_ref, pbuf, rs_buf, h_buf,
             rs_send, rs_recv, ag_send, ag_recv):
        my_pos = lax.axis_index("i")

        barrier_sem = pltpu.get_barrier_semaphore()
        for k in range(1, N_DEV):
            pl.semaphore_signal(
                barrier_sem, inc=1,
                device_id=((my_pos + k) % N_DEV,),
                device_id_type=pl.DeviceIdType.MESH,
            )
        pl.semaphore_wait(barrier_sem, N_DEV - 1)

        xv = x_ref[...]

        rs = []
        for k in range(1, N_DEV):
            peer = (my_pos + k) % N_DEV
            pbuf[k - 1] = jnp.dot(xv, win_ref[:, pl.ds(peer * q, q)],
                                  preferred_element_type=jnp.float32)
            rdma = pltpu.make_async_remote_copy(
                src_ref=pbuf.at[k - 1],
                dst_ref=rs_buf.at[k - 1],
                send_sem=rs_send.at[k - 1],
                recv_sem=rs_recv.at[k - 1],
                device_id=(peer,),
                device_id_type=pl.DeviceIdType.MESH,
            )
            rdma.start()
            rs.append(rdma)
        own_q = jnp.dot(xv, win_ref[:, pl.ds(my_pos * q, q)],
                        preferred_element_type=jnp.float32)
        for rdma in rs:
            rdma.wait()

        hq = jnp.maximum(own_q + rs_buf[0] + rs_buf[1] + rs_buf[2], 0.0)
        h_buf[:, pl.ds(my_pos * q, q)] = hq

        ag = []
        for k in range(1, N_DEV):
            peer = (my_pos + k) % N_DEV
            rdma = pltpu.make_async_remote_copy(
                src_ref=h_buf.at[:, pl.ds(my_pos * q, q)],
                dst_ref=h_buf.at[:, pl.ds(my_pos * q, q)],
                send_sem=ag_send.at[k - 1],
                recv_sem=ag_recv.at[k - 1],
                device_id=(peer,),
                device_id_type=pl.DeviceIdType.MESH,
            )
            rdma.start()
            ag.append(rdma)

        acc = jnp.dot(hq, wout_ref[pl.ds(my_pos * q, q), :],
                      preferred_element_type=jnp.float32)
        for rdma in ag:
            rdma.wait()
        for k in range(1, N_DEV):
            peer = (my_pos + k) % N_DEV
            acc = acc + jnp.dot(h_buf[:, pl.ds(peer * q, q)],
                                wout_ref[pl.ds(peer * q, q), :],
                                preferred_element_type=jnp.float32)
        out_ref[...] = acc

    return pl.pallas_call(
        body,
        out_shape=jax.ShapeDtypeStruct((b, d_shard), jnp.float32),
        in_specs=[pl.BlockSpec(memory_space=pltpu.VMEM)] * 3,
        out_specs=pl.BlockSpec(memory_space=pltpu.VMEM),
        scratch_shapes=[
            pltpu.VMEM((N_DEV - 1, b, q), jnp.float32),
            pltpu.VMEM((N_DEV - 1, b, q), jnp.float32),
            pltpu.VMEM((b, hdim), jnp.float32),
            pltpu.SemaphoreType.DMA((N_DEV - 1,)),
            pltpu.SemaphoreType.DMA((N_DEV - 1,)),
            pltpu.SemaphoreType.DMA((N_DEV - 1,)),
            pltpu.SemaphoreType.DMA((N_DEV - 1,)),
        ],
        compiler_params=pltpu.CompilerParams(collective_id=collective_id),
    )(x, win, wout)


def kernel(x, Win0, Wout0, Win1, Wout1, Win2, Wout2):
    x = _layer(x, Win0, Wout0, collective_id=0)
    x = _layer(x, Win1, Wout1, collective_id=1)
    x = _layer(x, Win2, Wout2, collective_id=2)
    return x
